# baseline (speedup 1.0000x reference)
import functools
import jax
import jax.numpy as jnp
from jax import lax
from jax.experimental import pallas as pl
from jax.experimental.pallas import tpu as pltpu
from jax.experimental.pallas import tpu_sc as plsc

N = 10000
E = 320000
H = 128
G = 256
L = 3

NC = 2
NS = 16
NW = NC * NS
EPW = E // NW
CK = 80
NCHUNK = EPW // CK
NPAD = 10240
RPS = NPAD // NS

_INTERPRET = False


def _embed_body(xf_ref, a0_ref, a1_ref, wa_ref, ba_ref,
                b0_ref, b1_ref, wb_ref, bb_ref, src2_ref,
                ea0_ref, ea1_ref, ea2_ref,
                h0_ref, lut_ref, eidx_ref):
    eidx_ref[...] = (src2_ref[...] * 8 + ea0_ref[...]
                     + 2 * ea1_ref[...] + 4 * ea2_ref[...])
    a0 = a0_ref[...]
    da = a1_ref[...] - a0
    base_a = jnp.sum(a0, axis=0, keepdims=True)
    t = jnp.dot(xf_ref[...], da, preferred_element_type=jnp.float32, precision=lax.Precision.HIGHEST) + base_a
    h0_ref[...] = jnp.dot(t, wa_ref[...], preferred_element_type=jnp.float32) + ba_ref[...]

    c = lax.broadcasted_iota(jnp.int32, (8, 3), 0)
    j = lax.broadcasted_iota(jnp.int32, (8, 3), 1)
    bits = ((c >> j) & 1).astype(jnp.float32)
    b0 = b0_ref[...]
    db = b1_ref[...] - b0
    base_b = jnp.sum(b0, axis=0, keepdims=True)
    te = jnp.dot(bits, db, preferred_element_type=jnp.float32, precision=lax.Precision.HIGHEST) + base_b
    lut_ref[...] = jnp.dot(te, wb_ref[...], preferred_element_type=jnp.float32) + bb_ref[...]


def _embed(xf, a0, a1, wa, ba, b0, b1, wb, bb, src2, ea0, ea1, ea2):
    return pl.pallas_call(
        _embed_body,
        out_shape=(jax.ShapeDtypeStruct((N, H), jnp.float32),
                   jax.ShapeDtypeStruct((8, H), jnp.float32),
                   jax.ShapeDtypeStruct((E // H, H), jnp.int32)),
        interpret=_INTERPRET,
    )(xf, a0, a1, wa, ba, b0, b1, wb, bb, src2, ea0, ea1, ea2)


def _bn(z, g, b):
    mu = jnp.mean(z, axis=0, keepdims=True)
    xc = z - mu
    var = jnp.mean(xc * xc, axis=0, keepdims=True)
    return xc / jnp.sqrt(var + 1e-5) * g + b


def _node_body(h_ref, parts_ref, eps_ref, w1_ref, b1_ref, g1_ref, be1_ref,
               w2_ref, b2_ref, g2_ref, be2_ref, out_ref):
    aggr = parts_ref[0, :N] + parts_ref[1, :N]
    z = (1.0 + eps_ref[0, 0]) * h_ref[...] + aggr
    z1 = jnp.dot(z, w1_ref[...], preferred_element_type=jnp.float32) + b1_ref[...]
    z1 = jnp.maximum(_bn(z1, g1_ref[...], be1_ref[...]), 0.0)
    z2 = jnp.dot(z1, w2_ref[...], preferred_element_type=jnp.float32) + b2_ref[...]
    out_ref[...] = jnp.maximum(_bn(z2, g2_ref[...], be2_ref[...]), 0.0)


def _node(h, parts, eps, w1, b1, g1, be1, w2, b2, g2, be2):
    return pl.pallas_call(
        _node_body,
        out_shape=jax.ShapeDtypeStruct((N, H), jnp.float32),
        interpret=_INTERPRET,
    )(h, parts, eps, w1, b1, g1, be1, w2, b2, g2, be2)


def _node_pool_body(h_ref, parts_ref, eps_ref, w1_ref, b1_ref, g1_ref,
                    be1_ref, w2_ref, b2_ref, g2_ref, be2_ref,
                    w_ref, b_ref, frag_ref, out_ref):
    aggr = parts_ref[0, :N] + parts_ref[1, :N]
    z = (1.0 + eps_ref[0, 0]) * h_ref[...] + aggr
    z1 = jnp.dot(z, w1_ref[...], preferred_element_type=jnp.float32) + b1_ref[...]
    z1 = jnp.maximum(_bn(z1, g1_ref[...], be1_ref[...]), 0.0)
    z2 = jnp.dot(z1, w2_ref[...], preferred_element_type=jnp.float32) + b2_ref[...]
    h3 = jnp.maximum(_bn(z2, g2_ref[...], be2_ref[...]), 0.0)
    hf = jnp.dot(h3, w_ref[...], preferred_element_type=jnp.float32) + b_ref[...]
    gids = lax.broadcasted_iota(jnp.int32, (1, G), 1)
    onehot = (frag_ref[...] == gids).astype(jnp.float32)
    out_ref[...] = lax.dot_general(onehot, hf, (((0,), (0,)), ((), ())),
                                   preferred_element_type=jnp.float32,
                                   precision=lax.Precision.HIGHEST)


def _node_pool(h, parts, eps, w1, b1, g1, be1, w2, b2, g2, be2, w, b, frag2d):
    return pl.pallas_call(
        _node_pool_body,
        out_shape=jax.ShapeDtypeStruct((G, H), jnp.float32),
        interpret=_INTERPRET,
    )(h, parts, eps, w1, b1, g1, be1, w2, b2, g2, be2, w, b, frag2d)


_MSG_BLK = 1000


def _msg8_body(h_ref, lut_ref, out_ref):
    h = h_ref[...]
    lut = lut_ref[...]
    m = jnp.maximum(h[:, None, :] + lut[None, :, :], 0.0)
    out_ref[...] = m.reshape(_MSG_BLK * 8, H)


def _msg8(h, lut):
    return pl.pallas_call(
        _msg8_body,
        grid=(N // _MSG_BLK,),
        in_specs=[pl.BlockSpec((_MSG_BLK, H), lambda i: (i, 0)),
                  pl.BlockSpec((8, H), lambda i: (0, 0))],
        out_specs=pl.BlockSpec((_MSG_BLK * 8, H), lambda i: (i, 0)),
        out_shape=jax.ShapeDtypeStruct((N * 8, H), jnp.float32),
        interpret=_INTERPRET,
    )(h, lut)


SUP = 2000
NSUP = EPW // SUP
CPS = SUP // CK


def _make_edge_call():
    mesh = plsc.VectorSubcoreMesh(core_axis_name="c", subcore_axis_name="s",
                                  num_cores=NC, num_subcores=NS)
    return pl.kernel(
        _edge_body,
        out_type=jax.ShapeDtypeStruct((NC, NPAD, H), jnp.float32),
        mesh=mesh,
        scratch_types=[
            pltpu.VMEM((SUP,), jnp.int32),
            pltpu.VMEM((CPS, CK), jnp.int32),
            pltpu.VMEM((CK, H), jnp.float32),
            pltpu.VMEM((CK, H), jnp.float32),
            pltpu.VMEM((CK, H), jnp.float32),
            pltpu.VMEM((CK, H), jnp.float32),
            pltpu.VMEM_SHARED((NPAD, H), jnp.float32),
            pltpu.SemaphoreType.DMA,
            pltpu.SemaphoreType.DMA,
            pltpu.SemaphoreType.DMA,
            pltpu.SemaphoreType.DMA,
            pltpu.SemaphoreType.DMA,
            pltpu.SemaphoreType.DMA,
            pltpu.SemaphoreType.DMA,
            pltpu.SemaphoreType.DMA,
        ],
    )


def _edge_body(msg8_hbm, eidxR_hbm, dstR_hbm, zeros_hbm, out_hbm,
               eidx_c, dst_c, rows_a, rows_b, rows_c, rows_d, aggr_sh,
               ga, gb, gc, gd, sa, sb, sc2, sd):
    cid = lax.axis_index("c")
    sid = lax.axis_index("s")
    wid = cid * NS + sid

    def _gather(jc, buf, sem):
        return pltpu.async_copy(msg8_hbm.at[eidx_c.at[pl.ds(jc * CK, CK)]],
                                buf, sem)

    def _scatter(jc, buf, sem):
        return pltpu.async_copy(buf, aggr_sh.at[dst_c.at[jc]], sem, add=True)

    def _wait_gather(buf, sem):
        pltpu.make_async_copy(msg8_hbm.at[pl.ds(0, CK)], buf, sem).wait()

    def _wait_scatter(buf, sem):
        pltpu.make_async_copy(buf, aggr_sh.at[pl.ds(0, CK)], sem).wait()

    pltpu.sync_copy(zeros_hbm.at[pl.ds(sid * RPS, RPS)],
                    aggr_sh.at[pl.ds(sid * RPS, RPS)])
    plsc.subcore_barrier()

    for sc in range(NSUP):
        base = wid * EPW + sc * SUP
        pltpu.sync_copy(eidxR_hbm.at[pl.ds(base, SUP)], eidx_c)
        pltpu.sync_copy(dstR_hbm.at[wid, sc], dst_c)

        NT = (CPS - 1) // 4
        _gather(0, rows_a, ga).wait()
        pltpu.sync_copy(rows_a, aggr_sh.at[dst_c.at[0]], add=True)
        _gather(1, rows_a, ga)
        _gather(2, rows_b, gb)
        _gather(3, rows_c, gc)

        def _quad(t, carry):
            c = 1 + 4 * t
            _wait_gather(rows_a, ga)
            _scatter(c, rows_a, sa)

            @pl.when(t > 0)
            def _():
                _wait_scatter(rows_d, sd)

            _gather(c + 3, rows_d, gd)
            _wait_gather(rows_b, gb)
            _scatter(c + 1, rows_b, sb)
            _wait_scatter(rows_a, sa)

            @pl.when(t < NT - 1)
            def _():
                _gather(c + 4, rows_a, ga)

            _wait_gather(rows_c, gc)
            _scatter(c + 2, rows_c, sc2)
            _wait_scatter(rows_b, sb)

            @pl.when(t < NT - 1)
            def _():
                _gather(c + 5, rows_b, gb)

            _wait_gather(rows_d, gd)
            _scatter(c + 3, rows_d, sd)
            _wait_scatter(rows_c, sc2)

            @pl.when(t < NT - 1)
            def _():
                _gather(c + 6, rows_c, gc)

            return carry

        lax.fori_loop(0, NT, _quad, 0)
        _wait_scatter(rows_d, sd)

    plsc.subcore_barrier()

    pltpu.sync_copy(aggr_sh.at[pl.ds(sid * RPS, RPS)],
                    out_hbm.at[cid, pl.ds(sid * RPS, RPS)])


def _edge_pass(msg8, eidx, dstR, zeros):
    return _make_edge_call()(msg8, eidx, dstR, zeros)


def kernel(x, edge_index, edge_attr, frag_batch, atom_tables, bond_tables,
           atom_lin_w, atom_lin_b, bond_lin_w, bond_lin_b,
           mlp_w1, mlp_b1, bn1_g, bn1_b, mlp_w2, mlp_b2,
           eps_conv, norm_g, norm_b, lin_w, lin_b):
    xf = x.astype(jnp.float32)
    src2 = edge_index[0].reshape(E // H, H)
    ea0 = edge_attr[:, 0].reshape(E // H, H)
    ea1 = edge_attr[:, 1].reshape(E // H, H)
    ea2 = edge_attr[:, 2].reshape(E // H, H)
    dstR = edge_index[1].reshape(NW, NSUP, CPS, CK)
    zeros = jnp.zeros((NPAD, H), jnp.float32)

    h, lut, eidx2 = _embed(xf,
                           atom_tables[:, 0, :], atom_tables[:, 1, :],
                           atom_lin_w, atom_lin_b.reshape(1, H),
                           bond_tables[:, 0, :], bond_tables[:, 1, :],
                           bond_lin_w, bond_lin_b.reshape(1, H),
                           src2, ea0, ea1, ea2)
    eidx = eidx2.reshape(E)

    for l in range(L - 1):
        msg8 = _msg8(h, lut)
        parts = _edge_pass(msg8, eidx, dstR, zeros)
        h = _node(h, parts, eps_conv[l].reshape(1, 1),
                  mlp_w1[l], mlp_b1[l].reshape(1, 2 * H),
                  bn1_g[l].reshape(1, 2 * H), bn1_b[l].reshape(1, 2 * H),
                  mlp_w2[l], mlp_b2[l].reshape(1, H),
                  norm_g[l].reshape(1, H), norm_b[l].reshape(1, H))

    l = L - 1
    msg8 = _msg8(h, lut)
    parts = _edge_pass(msg8, eidx, dstR, zeros)
    return _node_pool(h, parts, eps_conv[l].reshape(1, 1),
                      mlp_w1[l], mlp_b1[l].reshape(1, 2 * H),
                      bn1_g[l].reshape(1, 2 * H), bn1_b[l].reshape(1, 2 * H),
                      mlp_w2[l], mlp_b2[l].reshape(1, H),
                      norm_g[l].reshape(1, H), norm_b[l].reshape(1, H),
                      lin_w, lin_b.reshape(1, H), frag_batch.reshape(N, 1))


# speedup vs baseline: 1.0178x; 1.0178x over previous
import functools
import jax
import jax.numpy as jnp
from jax import lax
from jax.experimental import pallas as pl
from jax.experimental.pallas import tpu as pltpu
from jax.experimental.pallas import tpu_sc as plsc

N = 10000
E = 320000
H = 128
G = 256
L = 3

NC = 2
NS = 16
NW = NC * NS
EPW = E // NW
CK = 80
NCHUNK = EPW // CK
NPAD = 10240
RPS = NPAD // NS

_INTERPRET = False


def _embed_body(xf_ref, a0_ref, a1_ref, wa_ref, ba_ref,
                b0_ref, b1_ref, wb_ref, bb_ref, src2_ref,
                ea0_ref, ea1_ref, ea2_ref,
                h0_ref, lut_ref, eidx_ref):
    eidx_ref[...] = (src2_ref[...] * 8 + ea0_ref[...]
                     + 2 * ea1_ref[...] + 4 * ea2_ref[...])
    a0 = a0_ref[...]
    da = a1_ref[...] - a0
    base_a = jnp.sum(a0, axis=0, keepdims=True)
    t = jnp.dot(xf_ref[...], da, preferred_element_type=jnp.float32, precision=lax.Precision.HIGHEST) + base_a
    h0_ref[...] = jnp.dot(t, wa_ref[...], preferred_element_type=jnp.float32) + ba_ref[...]

    c = lax.broadcasted_iota(jnp.int32, (8, 3), 0)
    j = lax.broadcasted_iota(jnp.int32, (8, 3), 1)
    bits = ((c >> j) & 1).astype(jnp.float32)
    b0 = b0_ref[...]
    db = b1_ref[...] - b0
    base_b = jnp.sum(b0, axis=0, keepdims=True)
    te = jnp.dot(bits, db, preferred_element_type=jnp.float32, precision=lax.Precision.HIGHEST) + base_b
    lut_ref[...] = jnp.dot(te, wb_ref[...], preferred_element_type=jnp.float32) + bb_ref[...]


def _embed(xf, a0, a1, wa, ba, b0, b1, wb, bb, src2, ea0, ea1, ea2):
    return pl.pallas_call(
        _embed_body,
        out_shape=(jax.ShapeDtypeStruct((N, H), jnp.float32),
                   jax.ShapeDtypeStruct((8, H), jnp.float32),
                   jax.ShapeDtypeStruct((E // H, H), jnp.int32)),
        interpret=_INTERPRET,
    )(xf, a0, a1, wa, ba, b0, b1, wb, bb, src2, ea0, ea1, ea2)


def _bn(z, g, b):
    mu = jnp.mean(z, axis=0, keepdims=True)
    xc = z - mu
    var = jnp.mean(xc * xc, axis=0, keepdims=True)
    return xc / jnp.sqrt(var + 1e-5) * g + b


def _node_body(h_ref, parts_ref, eps_ref, w1_ref, b1_ref, g1_ref, be1_ref,
               w2_ref, b2_ref, g2_ref, be2_ref, out_ref):
    aggr = parts_ref[0, :N] + parts_ref[1, :N]
    z = (1.0 + eps_ref[0, 0]) * h_ref[...] + aggr
    z1 = jnp.dot(z, w1_ref[...], preferred_element_type=jnp.float32) + b1_ref[...]
    z1 = jnp.maximum(_bn(z1, g1_ref[...], be1_ref[...]), 0.0)
    z2 = jnp.dot(z1, w2_ref[...], preferred_element_type=jnp.float32) + b2_ref[...]
    out_ref[...] = jnp.maximum(_bn(z2, g2_ref[...], be2_ref[...]), 0.0)


def _node(h, parts, eps, w1, b1, g1, be1, w2, b2, g2, be2):
    return pl.pallas_call(
        _node_body,
        out_shape=jax.ShapeDtypeStruct((N, H), jnp.float32),
        interpret=_INTERPRET,
    )(h, parts, eps, w1, b1, g1, be1, w2, b2, g2, be2)


def _node_pool_body(h_ref, parts_ref, eps_ref, w1_ref, b1_ref, g1_ref,
                    be1_ref, w2_ref, b2_ref, g2_ref, be2_ref,
                    w_ref, b_ref, frag_ref, out_ref):
    aggr = parts_ref[0, :N] + parts_ref[1, :N]
    z = (1.0 + eps_ref[0, 0]) * h_ref[...] + aggr
    z1 = jnp.dot(z, w1_ref[...], preferred_element_type=jnp.float32) + b1_ref[...]
    z1 = jnp.maximum(_bn(z1, g1_ref[...], be1_ref[...]), 0.0)
    z2 = jnp.dot(z1, w2_ref[...], preferred_element_type=jnp.float32) + b2_ref[...]
    h3 = jnp.maximum(_bn(z2, g2_ref[...], be2_ref[...]), 0.0)
    hf = jnp.dot(h3, w_ref[...], preferred_element_type=jnp.float32) + b_ref[...]
    gids = lax.broadcasted_iota(jnp.int32, (1, G), 1)
    onehot = (frag_ref[...] == gids).astype(jnp.float32)
    out_ref[...] = lax.dot_general(onehot, hf, (((0,), (0,)), ((), ())),
                                   preferred_element_type=jnp.float32,
                                   precision=lax.Precision.HIGHEST)


def _node_pool(h, parts, eps, w1, b1, g1, be1, w2, b2, g2, be2, w, b, frag2d):
    return pl.pallas_call(
        _node_pool_body,
        out_shape=jax.ShapeDtypeStruct((G, H), jnp.float32),
        interpret=_INTERPRET,
    )(h, parts, eps, w1, b1, g1, be1, w2, b2, g2, be2, w, b, frag2d)


_MSG_BLK = 1000


def _msg8_body(h_ref, lut_ref, out_ref):
    h = h_ref[...]
    lut = lut_ref[...]
    m = jnp.maximum(h[:, None, :] + lut[None, :, :], 0.0)
    out_ref[...] = m.reshape(_MSG_BLK * 8, H)


def _msg8(h, lut):
    return pl.pallas_call(
        _msg8_body,
        grid=(N // _MSG_BLK,),
        in_specs=[pl.BlockSpec((_MSG_BLK, H), lambda i: (i, 0)),
                  pl.BlockSpec((8, H), lambda i: (0, 0))],
        out_specs=pl.BlockSpec((_MSG_BLK * 8, H), lambda i: (i, 0)),
        out_shape=jax.ShapeDtypeStruct((N * 8, H), jnp.float32),
        interpret=_INTERPRET,
    )(h, lut)


SUP = 2000
NSUP = EPW // SUP
CPS = SUP // CK


def _make_edge_call():
    mesh = plsc.VectorSubcoreMesh(core_axis_name="c", subcore_axis_name="s",
                                  num_cores=NC, num_subcores=NS)
    return pl.kernel(
        _edge_body,
        out_type=jax.ShapeDtypeStruct((NC, NPAD, H), jnp.float32),
        mesh=mesh,
        scratch_types=[
            pltpu.VMEM((SUP,), jnp.int32),
            pltpu.VMEM((CPS, CK), jnp.int32),
            pltpu.VMEM((CK, H), jnp.float32),
            pltpu.VMEM((CK, H), jnp.float32),
            pltpu.VMEM((CK, H), jnp.float32),
            pltpu.VMEM_SHARED((NPAD, H), jnp.float32),
            pltpu.SemaphoreType.DMA,
            pltpu.SemaphoreType.DMA,
            pltpu.SemaphoreType.DMA,
            pltpu.SemaphoreType.DMA,
            pltpu.SemaphoreType.DMA,
            pltpu.SemaphoreType.DMA,
        ],
    )


def _edge_body(msg8_hbm, eidxR_hbm, dstR_hbm, zeros_hbm, out_hbm,
               eidx_c, dst_c, rows_a, rows_b, rows_c, aggr_sh,
               ga, gb, gc, sa, sb, sc2):
    cid = lax.axis_index("c")
    sid = lax.axis_index("s")
    wid = cid * NS + sid

    def _gather(jc, buf, sem):
        return pltpu.async_copy(msg8_hbm.at[eidx_c.at[pl.ds(jc * CK, CK)]],
                                buf, sem)

    def _scatter(jc, buf, sem):
        return pltpu.async_copy(buf, aggr_sh.at[dst_c.at[jc]], sem, add=True)

    def _wait_gather(buf, sem):
        pltpu.make_async_copy(msg8_hbm.at[pl.ds(0, CK)], buf, sem).wait()

    def _wait_scatter(buf, sem):
        pltpu.make_async_copy(buf, aggr_sh.at[pl.ds(0, CK)], sem).wait()

    pltpu.sync_copy(zeros_hbm.at[pl.ds(sid * RPS, RPS)],
                    aggr_sh.at[pl.ds(sid * RPS, RPS)])
    plsc.subcore_barrier()

    for sc in range(NSUP):
        base = wid * EPW + sc * SUP
        pltpu.sync_copy(eidxR_hbm.at[pl.ds(base, SUP)], eidx_c)
        pltpu.sync_copy(dstR_hbm.at[wid, sc], dst_c)

        NT = (CPS - 1) // 3
        _gather(0, rows_a, ga).wait()
        pltpu.sync_copy(rows_a, aggr_sh.at[dst_c.at[0]], add=True)
        _gather(1, rows_a, ga)
        _gather(2, rows_b, gb)

        def _triplet(t, carry):
            c = 1 + 3 * t
            _wait_gather(rows_a, ga)
            _scatter(c, rows_a, sa)

            @pl.when(t > 0)
            def _():
                _wait_scatter(rows_c, sc2)

            _gather(c + 2, rows_c, gc)
            _wait_gather(rows_b, gb)
            _scatter(c + 1, rows_b, sb)
            _wait_scatter(rows_a, sa)

            @pl.when(t < NT - 1)
            def _():
                _gather(c + 3, rows_a, ga)

            _wait_gather(rows_c, gc)
            _scatter(c + 2, rows_c, sc2)
            _wait_scatter(rows_b, sb)

            @pl.when(t < NT - 1)
            def _():
                _gather(c + 4, rows_b, gb)

            return carry

        lax.fori_loop(0, NT, _triplet, 0)
        _wait_scatter(rows_c, sc2)

    plsc.subcore_barrier()

    pltpu.sync_copy(aggr_sh.at[pl.ds(sid * RPS, RPS)],
                    out_hbm.at[cid, pl.ds(sid * RPS, RPS)])


def _edge_pass(msg8, eidx, dstR, zeros):
    return _make_edge_call()(msg8, eidx, dstR, zeros)


def kernel(x, edge_index, edge_attr, frag_batch, atom_tables, bond_tables,
           atom_lin_w, atom_lin_b, bond_lin_w, bond_lin_b,
           mlp_w1, mlp_b1, bn1_g, bn1_b, mlp_w2, mlp_b2,
           eps_conv, norm_g, norm_b, lin_w, lin_b):
    xf = x.astype(jnp.float32)
    src2 = edge_index[0].reshape(E // H, H)
    ea0 = edge_attr[:, 0].reshape(E // H, H)
    ea1 = edge_attr[:, 1].reshape(E // H, H)
    ea2 = edge_attr[:, 2].reshape(E // H, H)
    dstR = edge_index[1].reshape(NW, NSUP, CPS, CK)
    zeros = jnp.zeros((NPAD, H), jnp.float32)

    h, lut, eidx2 = _embed(xf,
                           atom_tables[:, 0, :], atom_tables[:, 1, :],
                           atom_lin_w, atom_lin_b.reshape(1, H),
                           bond_tables[:, 0, :], bond_tables[:, 1, :],
                           bond_lin_w, bond_lin_b.reshape(1, H),
                           src2, ea0, ea1, ea2)
    eidx = eidx2.reshape(E)

    for l in range(L - 1):
        msg8 = _msg8(h, lut)
        parts = _edge_pass(msg8, eidx, dstR, zeros)
        h = _node(h, parts, eps_conv[l].reshape(1, 1),
                  mlp_w1[l], mlp_b1[l].reshape(1, 2 * H),
                  bn1_g[l].reshape(1, 2 * H), bn1_b[l].reshape(1, 2 * H),
                  mlp_w2[l], mlp_b2[l].reshape(1, H),
                  norm_g[l].reshape(1, H), norm_b[l].reshape(1, H))

    l = L - 1
    msg8 = _msg8(h, lut)
    parts = _edge_pass(msg8, eidx, dstR, zeros)
    return _node_pool(h, parts, eps_conv[l].reshape(1, 1),
                      mlp_w1[l], mlp_b1[l].reshape(1, 2 * H),
                      bn1_g[l].reshape(1, 2 * H), bn1_b[l].reshape(1, 2 * H),
                      mlp_w2[l], mlp_b2[l].reshape(1, H),
                      norm_g[l].reshape(1, H), norm_b[l].reshape(1, H),
                      lin_w, lin_b.reshape(1, H), frag_batch.reshape(N, 1))


# speedup vs baseline: 1.0180x; 1.0001x over previous
import jax
import jax.numpy as jnp
from jax import lax
from jax.experimental import pallas as pl
from jax.experimental.pallas import tpu as pltpu
from jax.experimental.pallas import tpu_sc as plsc

N = 10000
E = 320000
H = 128
G = 256
L = 3

NC = 2
NS = 16
NW = NC * NS
EPW = E // NW
CK = 80
NPAD = 10240
RPS = NPAD // NS


def _embed_body(xf_ref, a0_ref, a1_ref, wa_ref, ba_ref,
                b0_ref, b1_ref, wb_ref, bb_ref, src2_ref,
                ea0_ref, ea1_ref, ea2_ref,
                h0_ref, lut_ref, eidx_ref):
    eidx_ref[...] = (src2_ref[...] * 8 + ea0_ref[...]
                     + 2 * ea1_ref[...] + 4 * ea2_ref[...])
    a0 = a0_ref[...]
    da = a1_ref[...] - a0
    base_a = jnp.sum(a0, axis=0, keepdims=True)
    t = jnp.dot(xf_ref[...], da, preferred_element_type=jnp.float32, precision=lax.Precision.HIGHEST) + base_a
    h0_ref[...] = jnp.dot(t, wa_ref[...], preferred_element_type=jnp.float32) + ba_ref[...]

    c = lax.broadcasted_iota(jnp.int32, (8, 3), 0)
    j = lax.broadcasted_iota(jnp.int32, (8, 3), 1)
    bits = ((c >> j) & 1).astype(jnp.float32)
    b0 = b0_ref[...]
    db = b1_ref[...] - b0
    base_b = jnp.sum(b0, axis=0, keepdims=True)
    te = jnp.dot(bits, db, preferred_element_type=jnp.float32, precision=lax.Precision.HIGHEST) + base_b
    lut_ref[...] = jnp.dot(te, wb_ref[...], preferred_element_type=jnp.float32) + bb_ref[...]


def _embed(xf, a0, a1, wa, ba, b0, b1, wb, bb, src2, ea0, ea1, ea2):
    return pl.pallas_call(
        _embed_body,
        out_shape=(jax.ShapeDtypeStruct((N, H), jnp.float32),
                   jax.ShapeDtypeStruct((8, H), jnp.float32),
                   jax.ShapeDtypeStruct((E // H, H), jnp.int32)),
    )(xf, a0, a1, wa, ba, b0, b1, wb, bb, src2, ea0, ea1, ea2)


def _bn(z, g, b):
    mu = jnp.mean(z, axis=0, keepdims=True)
    xc = z - mu
    var = jnp.mean(xc * xc, axis=0, keepdims=True)
    return xc / jnp.sqrt(var + 1e-5) * g + b


def _node_body(h_ref, parts_ref, eps_ref, w1_ref, b1_ref, g1_ref, be1_ref,
               w2_ref, b2_ref, g2_ref, be2_ref, out_ref):
    aggr = parts_ref[0, :N] + parts_ref[1, :N]
    z = (1.0 + eps_ref[0, 0]) * h_ref[...] + aggr
    z1 = jnp.dot(z, w1_ref[...], preferred_element_type=jnp.float32) + b1_ref[...]
    z1 = jnp.maximum(_bn(z1, g1_ref[...], be1_ref[...]), 0.0)
    z2 = jnp.dot(z1, w2_ref[...], preferred_element_type=jnp.float32) + b2_ref[...]
    out_ref[...] = jnp.maximum(_bn(z2, g2_ref[...], be2_ref[...]), 0.0)


def _node(h, parts, eps, w1, b1, g1, be1, w2, b2, g2, be2):
    return pl.pallas_call(
        _node_body,
        out_shape=jax.ShapeDtypeStruct((N, H), jnp.float32),
    )(h, parts, eps, w1, b1, g1, be1, w2, b2, g2, be2)


def _node_pool_body(h_ref, parts_ref, eps_ref, w1_ref, b1_ref, g1_ref,
                    be1_ref, w2_ref, b2_ref, g2_ref, be2_ref,
                    w_ref, b_ref, frag_ref, out_ref):
    aggr = parts_ref[0, :N] + parts_ref[1, :N]
    z = (1.0 + eps_ref[0, 0]) * h_ref[...] + aggr
    z1 = jnp.dot(z, w1_ref[...], preferred_element_type=jnp.float32) + b1_ref[...]
    z1 = jnp.maximum(_bn(z1, g1_ref[...], be1_ref[...]), 0.0)
    z2 = jnp.dot(z1, w2_ref[...], preferred_element_type=jnp.float32) + b2_ref[...]
    h3 = jnp.maximum(_bn(z2, g2_ref[...], be2_ref[...]), 0.0)
    hf = jnp.dot(h3, w_ref[...], preferred_element_type=jnp.float32) + b_ref[...]
    gids = lax.broadcasted_iota(jnp.int32, (1, G), 1)
    onehot = (frag_ref[...] == gids).astype(jnp.float32)
    out_ref[...] = lax.dot_general(onehot, hf, (((0,), (0,)), ((), ())),
                                   preferred_element_type=jnp.float32,
                                   precision=lax.Precision.HIGHEST)


def _node_pool(h, parts, eps, w1, b1, g1, be1, w2, b2, g2, be2, w, b, frag2d):
    return pl.pallas_call(
        _node_pool_body,
        out_shape=jax.ShapeDtypeStruct((G, H), jnp.float32),
    )(h, parts, eps, w1, b1, g1, be1, w2, b2, g2, be2, w, b, frag2d)


_MSG_BLK = 1000


def _msg8_body(h_ref, lut_ref, out_ref):
    h = h_ref[...]
    lut = lut_ref[...]
    m = jnp.maximum(h[:, None, :] + lut[None, :, :], 0.0)
    out_ref[...] = m.reshape(_MSG_BLK * 8, H)


def _msg8(h, lut):
    return pl.pallas_call(
        _msg8_body,
        grid=(N // _MSG_BLK,),
        in_specs=[pl.BlockSpec((_MSG_BLK, H), lambda i: (i, 0)),
                  pl.BlockSpec((8, H), lambda i: (0, 0))],
        out_specs=pl.BlockSpec((_MSG_BLK * 8, H), lambda i: (i, 0)),
        out_shape=jax.ShapeDtypeStruct((N * 8, H), jnp.float32),
    )(h, lut)


SUP = 2000
NSUP = EPW // SUP
CPS = SUP // CK


def _make_edge_call():
    mesh = plsc.VectorSubcoreMesh(core_axis_name="c", subcore_axis_name="s",
                                  num_cores=NC, num_subcores=NS)
    return pl.kernel(
        _edge_body,
        out_type=jax.ShapeDtypeStruct((NC, NPAD, H), jnp.float32),
        mesh=mesh,
        scratch_types=[
            pltpu.VMEM((SUP,), jnp.int32),
            pltpu.VMEM((CPS, CK), jnp.int32),
            pltpu.VMEM((CK, H), jnp.float32),
            pltpu.VMEM((CK, H), jnp.float32),
            pltpu.VMEM((CK, H), jnp.float32),
            pltpu.VMEM_SHARED((NPAD, H), jnp.float32),
            pltpu.SemaphoreType.DMA,
            pltpu.SemaphoreType.DMA,
            pltpu.SemaphoreType.DMA,
            pltpu.SemaphoreType.DMA,
            pltpu.SemaphoreType.DMA,
            pltpu.SemaphoreType.DMA,
        ],
    )


def _edge_body(msg8_hbm, eidxR_hbm, dstR_hbm, zeros_hbm, out_hbm,
               eidx_c, dst_c, rows_a, rows_b, rows_c, aggr_sh,
               ga, gb, gc, sa, sb, sc2):
    cid = lax.axis_index("c")
    sid = lax.axis_index("s")
    wid = cid * NS + sid

    def _gather(jc, buf, sem):
        return pltpu.async_copy(msg8_hbm.at[eidx_c.at[pl.ds(jc * CK, CK)]],
                                buf, sem)

    def _scatter(jc, buf, sem):
        return pltpu.async_copy(buf, aggr_sh.at[dst_c.at[jc]], sem, add=True)

    def _wait_gather(buf, sem):
        pltpu.make_async_copy(msg8_hbm.at[pl.ds(0, CK)], buf, sem).wait()

    def _wait_scatter(buf, sem):
        pltpu.make_async_copy(buf, aggr_sh.at[pl.ds(0, CK)], sem).wait()

    pltpu.sync_copy(zeros_hbm.at[pl.ds(sid * RPS, RPS)],
                    aggr_sh.at[pl.ds(sid * RPS, RPS)])
    plsc.subcore_barrier()

    for sc in range(NSUP):
        base = wid * EPW + sc * SUP
        pltpu.sync_copy(eidxR_hbm.at[pl.ds(base, SUP)], eidx_c)
        pltpu.sync_copy(dstR_hbm.at[wid, sc], dst_c)

        NT = (CPS - 1) // 3
        _gather(0, rows_a, ga).wait()
        pltpu.sync_copy(rows_a, aggr_sh.at[dst_c.at[0]], add=True)
        _gather(1, rows_a, ga)
        _gather(2, rows_b, gb)

        def _triplet(t, carry):
            c = 1 + 3 * t
            _wait_gather(rows_a, ga)
            _scatter(c, rows_a, sa)

            @pl.when(t > 0)
            def _():
                _wait_scatter(rows_c, sc2)

            _gather(c + 2, rows_c, gc)
            _wait_gather(rows_b, gb)
            _scatter(c + 1, rows_b, sb)
            _wait_scatter(rows_a, sa)

            @pl.when(t < NT - 1)
            def _():
                _gather(c + 3, rows_a, ga)

            _wait_gather(rows_c, gc)
            _scatter(c + 2, rows_c, sc2)
            _wait_scatter(rows_b, sb)

            @pl.when(t < NT - 1)
            def _():
                _gather(c + 4, rows_b, gb)

            return carry

        lax.fori_loop(0, NT, _triplet, 0)
        _wait_scatter(rows_c, sc2)

    plsc.subcore_barrier()

    pltpu.sync_copy(aggr_sh.at[pl.ds(sid * RPS, RPS)],
                    out_hbm.at[cid, pl.ds(sid * RPS, RPS)])


def _edge_pass(msg8, eidx, dstR, zeros):
    return _make_edge_call()(msg8, eidx, dstR, zeros)


def kernel(x, edge_index, edge_attr, frag_batch, atom_tables, bond_tables,
           atom_lin_w, atom_lin_b, bond_lin_w, bond_lin_b,
           mlp_w1, mlp_b1, bn1_g, bn1_b, mlp_w2, mlp_b2,
           eps_conv, norm_g, norm_b, lin_w, lin_b):
    xf = x.astype(jnp.float32)
    src2 = edge_index[0].reshape(E // H, H)
    ea0 = edge_attr[:, 0].reshape(E // H, H)
    ea1 = edge_attr[:, 1].reshape(E // H, H)
    ea2 = edge_attr[:, 2].reshape(E // H, H)
    dstR = edge_index[1].reshape(NW, NSUP, CPS, CK)
    zeros = jnp.zeros((NPAD, H), jnp.float32)

    h, lut, eidx2 = _embed(xf,
                           atom_tables[:, 0, :], atom_tables[:, 1, :],
                           atom_lin_w, atom_lin_b.reshape(1, H),
                           bond_tables[:, 0, :], bond_tables[:, 1, :],
                           bond_lin_w, bond_lin_b.reshape(1, H),
                           src2, ea0, ea1, ea2)
    eidx = eidx2.reshape(E)

    for l in range(L - 1):
        msg8 = _msg8(h, lut)
        parts = _edge_pass(msg8, eidx, dstR, zeros)
        h = _node(h, parts, eps_conv[l].reshape(1, 1),
                  mlp_w1[l], mlp_b1[l].reshape(1, 2 * H),
                  bn1_g[l].reshape(1, 2 * H), bn1_b[l].reshape(1, 2 * H),
                  mlp_w2[l], mlp_b2[l].reshape(1, H),
                  norm_g[l].reshape(1, H), norm_b[l].reshape(1, H))

    l = L - 1
    msg8 = _msg8(h, lut)
    parts = _edge_pass(msg8, eidx, dstR, zeros)
    return _node_pool(h, parts, eps_conv[l].reshape(1, 1),
                      mlp_w1[l], mlp_b1[l].reshape(1, 2 * H),
                      bn1_g[l].reshape(1, 2 * H), bn1_b[l].reshape(1, 2 * H),
                      mlp_w2[l], mlp_b2[l].reshape(1, H),
                      norm_g[l].reshape(1, H), norm_b[l].reshape(1, H),
                      lin_w, lin_b.reshape(1, H), frag_batch.reshape(N, 1))
